# baseline (speedup 1.0000x reference)
import functools

import jax
import jax.numpy as jnp
from jax import lax
from jax.experimental import pallas as pl
from jax.experimental.pallas import tpu as pltpu
from jax.experimental.pallas import tpu_sc as plsc

_NW = 32
_W = 96
_WC = _W // 16


def _sc_body(vs2_hbm, rp_hbm, tari_hbm, tarj_hbm, x2d_hbm, xflat_hbm,
             xcn_hbm, xij_hbm,
             tari_v, tarj_v, tmp_v, li_v, ri_v, lj_v, rj_v,
             wm, xcnbuf, xibuf, xjbuf, xrow,
             idxbuf, segjbuf, segibuf, widxbuf, wsegbuf,
             sem0, sem1, sem2,
             *, PW: int, DIN: int, NSENT: int):
    wid = lax.axis_index("s") * 2 + lax.axis_index("c")
    base = wid * PW
    ii = lax.iota(jnp.int32, 16)
    zeros16 = jnp.zeros((16,), jnp.int32)
    ones16 = jnp.ones((16,), jnp.int32)
    DC = DIN // 16

    pltpu.sync_copy(tari_hbm.at[pl.ds(base, PW)], tari_v)
    pltpu.sync_copy(tarj_hbm.at[pl.ds(base, PW)], tarj_v)
    pltpu.async_copy(rp_hbm.at[tari_v], li_v, sem0).wait()
    pltpu.async_copy(rp_hbm.at[tarj_v], lj_v, sem0).wait()

    def _plus1(c, buf_src, buf_dst):
        buf_dst[pl.ds(c * 16, 16)] = buf_src[pl.ds(c * 16, 16)] + 1

    lax.fori_loop(0, PW // 16, lambda c, _: (_plus1(c, tari_v, tmp_v), 0)[1], 0)
    pltpu.async_copy(rp_hbm.at[tmp_v], ri_v, sem0).wait()
    lax.fori_loop(0, PW // 16, lambda c, _: (_plus1(c, tarj_v, tmp_v), 0)[1], 0)
    pltpu.async_copy(rp_hbm.at[tmp_v], rj_v, sem0).wait()

    def _scal(ref, p):
        cb = (p // 16) * 16
        vec = ref[pl.ds(cb, 16)]
        return jnp.sum(jnp.where(ii == (p - cb), vec, 0))

    pltpu.async_copy(x2d_hbm.at[tari_v], xibuf, sem0)
    pltpu.async_copy(x2d_hbm.at[tarj_v], xjbuf, sem1)
    pltpu.make_async_copy(x2d_hbm.at[tari_v], xibuf, sem0).wait()
    pltpu.make_async_copy(x2d_hbm.at[tarj_v], xjbuf, sem1).wait()

    def _mul(t, _):
        p = t // DC
        c = t % DC
        xibuf[p, pl.ds(c * 16, 16)] = (xibuf[p, pl.ds(c * 16, 16)]
                                       * xjbuf[p, pl.ds(c * 16, 16)])
        return 0

    lax.fori_loop(0, PW * DC, _mul, 0)
    pltpu.sync_copy(xibuf, xij_hbm.at[pl.ds(base, PW)])

    def _zix(k, _):
        idxbuf[pl.ds(k * 16, 16)] = zeros16
        return 0

    lax.fori_loop(0, (4 * _W) // 16, _zix, 0)

    NWM = wm.shape[0]

    def _zwm(k, _):
        wm[pl.ds(k * 16, 16)] = zeros16
        return 0

    lax.fori_loop(0, NWM // 16, _zwm, 0)

    def _zxc(t, _):
        p = t // DC
        c = t % DC
        xcnbuf[p, pl.ds(c * 16, 16)] = jnp.zeros((16,), jnp.float32)
        return 0

    lax.fori_loop(0, PW * DC, _zxc, 0)

    def _fire(p):
        sl = (p % 2) * _W
        l_j = _scal(lj_v, p)
        r_j = _scal(rj_v, p)
        l_i = _scal(li_v, p)
        r_i = _scal(ri_v, p)
        def _bld(c, _):
            idxj = jnp.maximum(0, jnp.minimum(l_j + c * 16 + ii, r_j - 1))
            idxbuf[pl.ds(sl + c * 16, 16)] = idxj
            idxi = jnp.maximum(0, jnp.minimum(l_i + c * 16 + ii, r_i - 1))
            idxbuf[pl.ds(2 * _W + sl + c * 16, 16)] = idxi
            return 0

        nch = jnp.minimum(_WC, (jnp.maximum(r_j - l_j, r_i - l_i) + 15) >> 4)
        lax.fori_loop(0, nch, _bld, 0)
        pltpu.async_copy(vs2_hbm.at[idxbuf.at[pl.ds(sl, _W)]],
                         segjbuf.at[pl.ds(sl, _W)], sem2)
        pltpu.async_copy(vs2_hbm.at[idxbuf.at[pl.ds(2 * _W + sl, _W)]],
                         segibuf.at[pl.ds(sl, _W)], sem2)

    def _drain(p):
        sl = (p % 2) * _W
        pltpu.make_async_copy(vs2_hbm.at[idxbuf.at[pl.ds(sl, _W)]],
                              segjbuf.at[pl.ds(sl, _W)], sem2).wait()
        pltpu.make_async_copy(vs2_hbm.at[idxbuf.at[pl.ds(2 * _W + sl, _W)]],
                              segibuf.at[pl.ds(sl, _W)], sem2).wait()

    def _set_chunk(vj, m):
        plsc.store_scatter(wm, [vj], ones16, mask=m)

    def _clear_chunk(vj, m):
        plsc.store_scatter(wm, [vj], zeros16, mask=m)

    def _probe_chunk(p, vi, m):
        hit = plsc.load_gather(wm, [vi])
        hv = jnp.where(m, hit, 0)
        anyhit = jnp.max(hv)

        @pl.when(anyhit > 0)
        def _():
            def _lane(lane, _):
                w_l = jnp.sum(jnp.where(ii == lane, hv, 0))

                @pl.when(w_l > 0)
                def _():
                    v_l = jnp.sum(jnp.where(ii == lane, vi, 0))
                    off = pl.multiple_of(v_l * DIN, DIN)
                    pltpu.sync_copy(xflat_hbm.at[pl.ds(off, DIN)], xrow)

                    def _acc(c2, _):
                        xcnbuf[p, pl.ds(c2 * 16, 16)] = (
                            xcnbuf[p, pl.ds(c2 * 16, 16)]
                            + xrow[pl.ds(c2 * 16, 16)])
                        return 0

                    lax.fori_loop(0, DC, _acc, 0)
                return 0

            lax.fori_loop(0, 16, _lane, 0)

    def _extra_windows(l0, r, chunk_fn):
        def cond(off):
            return off < r

        def step(off):
            def _bw(c, _):
                idx = jnp.maximum(0, jnp.minimum(off + c * 16 + ii, r - 1))
                widxbuf[pl.ds(c * 16, 16)] = idx
                return 0

            lax.fori_loop(0, _WC, _bw, 0)
            pltpu.async_copy(vs2_hbm.at[widxbuf], wsegbuf, sem0).wait()

            def _pw(c, _):
                vec = wsegbuf[pl.ds(c * 16, 16)]
                m = (off + c * 16 + ii) < r
                chunk_fn(vec, m)
                return 0

            lax.fori_loop(0, _WC, _pw, 0)
            return off + _W

        lax.while_loop(cond, step, l0 + _W)

    _fire(0)

    def _pair(p, _):
        _drain(p)

        @pl.when(p + 1 < PW)
        def _():
            _fire(p + 1)

        sl = (p % 2) * _W
        l_j = _scal(lj_v, p)
        r_j = _scal(rj_v, p)
        l_i = _scal(li_v, p)
        r_i = _scal(ri_v, p)

        def _sc(c, _):
            vj = segjbuf[pl.ds(sl + c * 16, 16)]
            m = (l_j + c * 16 + ii) < r_j
            _set_chunk(vj, m)
            return 0

        njc = jnp.minimum(_WC, (r_j - l_j + 15) >> 4)
        nic = jnp.minimum(_WC, (r_i - l_i + 15) >> 4)
        lax.fori_loop(0, njc, _sc, 0)

        @pl.when(l_j + _W < r_j)
        def _():
            _extra_windows(l_j, r_j, _set_chunk)

        def _pc(c, _):
            vi = segibuf[pl.ds(sl + c * 16, 16)]
            m = (l_i + c * 16 + ii) < r_i
            _probe_chunk(p, vi, m)
            return 0

        lax.fori_loop(0, nic, _pc, 0)

        @pl.when(l_i + _W < r_i)
        def _():
            _extra_windows(l_i, r_i, lambda vec, m: _probe_chunk(p, vec, m))

        def _cc(c, _):
            vj = segjbuf[pl.ds(sl + c * 16, 16)]
            m = (l_j + c * 16 + ii) < r_j
            _clear_chunk(vj, m)
            return 0

        lax.fori_loop(0, njc, _cc, 0)

        @pl.when(l_j + _W < r_j)
        def _():
            _extra_windows(l_j, r_j, _clear_chunk)

        return 0

    lax.fori_loop(0, PW, _pair, 0)

    pltpu.sync_copy(xcnbuf, xcn_hbm.at[pl.ds(base, PW)])


def _mlp_body(xcn_ref, xij_ref,
              Wcn1_r, bcn1_r, Wcn2_r, bcn2_r, Wcn3_r, bcn3_r,
              Wij1_r, bij1_r, Wij2_r, bij2_r,
              Wl1_r, bl1_r, Wl2_r, bl2_r, beta_r,
              out_ref):
    f32 = jnp.float32
    h = jnp.maximum(jnp.dot(xcn_ref[...], Wcn1_r[...],
                            preferred_element_type=f32) + bcn1_r[...], 0.0)
    h = jnp.maximum(jnp.dot(h, Wcn2_r[...],
                            preferred_element_type=f32) + bcn2_r[...], 0.0)
    hcn = jnp.dot(h, Wcn3_r[...], preferred_element_type=f32) + bcn3_r[...]

    hij = jnp.maximum(jnp.dot(xij_ref[...], Wij1_r[...],
                              preferred_element_type=f32) + bij1_r[...], 0.0)
    hij = jnp.dot(hij, Wij2_r[...], preferred_element_type=f32) + bij2_r[...]

    z = hcn * beta_r[0, 0] + hij
    o = jnp.maximum(jnp.dot(z, Wl1_r[...],
                            preferred_element_type=f32) + bl1_r[...], 0.0)
    out_ref[...] = (jnp.dot(o, Wl2_r[...], preferred_element_type=f32)
                    + bl2_r[0, 0])


def kernel(x, edge_index, tar_ei, beta, Wcn1, bcn1, Wcn2, bcn2, Wcn3, bcn3,
           Wij1, bij1, Wij2, bij2, Wl1, bl1, Wl2, bl2):
    N, DIN = x.shape
    E = edge_index.shape[1]
    B = tar_ei.shape[1]
    DH = Wcn1.shape[1]
    DOUT = Wl2.shape[1]
    PW = B // _NW

    M = 1
    while M < N:
        M *= 2
    e0 = edge_index[0].astype(jnp.int32)
    e1 = edge_index[1].astype(jnp.int32)
    keys = jnp.sort(e0 * M + e1)
    v = keys & (M - 1)
    uq = jnp.concatenate([jnp.ones((1,), bool), keys[1:] != keys[:-1]])
    vs2 = jnp.where(uq, v, N).astype(jnp.int32)
    deg = jnp.zeros((N,), jnp.int32).at[e0].add(1)
    rp = jnp.concatenate([jnp.zeros((1,), jnp.int32),
                          jnp.cumsum(deg).astype(jnp.int32)])
    tari = tar_ei[0].astype(jnp.int32)
    tarj = tar_ei[1].astype(jnp.int32)
    xflat = jnp.zeros(((N + 8) * DIN,), jnp.float32).at[:N * DIN].set(
        x.reshape(-1))
    NWM = ((N + 1 + 15) // 16) * 16

    sc = functools.partial(
        pl.kernel,
        mesh=plsc.VectorSubcoreMesh(core_axis_name="c", subcore_axis_name="s"),
        compiler_params=pltpu.CompilerParams(needs_layout_passes=False),
        out_type=[jax.ShapeDtypeStruct((B, DIN), jnp.float32),
                  jax.ShapeDtypeStruct((B, DIN), jnp.float32)],
        scratch_types=[
            pltpu.VMEM((PW,), jnp.int32),
            pltpu.VMEM((PW,), jnp.int32),
            pltpu.VMEM((PW,), jnp.int32),
            pltpu.VMEM((PW,), jnp.int32),
            pltpu.VMEM((PW,), jnp.int32),
            pltpu.VMEM((PW,), jnp.int32),
            pltpu.VMEM((PW,), jnp.int32),
            pltpu.VMEM((NWM,), jnp.int32),
            pltpu.VMEM((PW, DIN), jnp.float32),
            pltpu.VMEM((PW, DIN), jnp.float32),
            pltpu.VMEM((PW, DIN), jnp.float32),
            pltpu.VMEM((DIN,), jnp.float32),
            pltpu.VMEM((4 * _W,), jnp.int32),
            pltpu.VMEM((2 * _W,), jnp.int32),
            pltpu.VMEM((2 * _W,), jnp.int32),
            pltpu.VMEM((_W,), jnp.int32),
            pltpu.VMEM((_W,), jnp.int32),
            pltpu.SemaphoreType.DMA,
            pltpu.SemaphoreType.DMA,
            pltpu.SemaphoreType.DMA,
        ],
    )
    xcn, xij = sc(functools.partial(_sc_body, PW=PW, DIN=DIN, NSENT=N))(
        vs2, rp, tari, tarj, x, xflat)

    BBM = 512 if B % 512 == 0 else B
    full = lambda shape: pl.BlockSpec(shape, lambda g: (0, 0))
    out = pl.pallas_call(
        _mlp_body,
        grid=(B // BBM,),
        in_specs=[
            pl.BlockSpec((BBM, DIN), lambda g: (g, 0)),
            pl.BlockSpec((BBM, DIN), lambda g: (g, 0)),
            full((DIN, DH)), full((1, DH)),
            full((DH, DH)), full((1, DH)),
            full((DH, DH)), full((1, DH)),
            full((DIN, DH)), full((1, DH)),
            full((DH, DH)), full((1, DH)),
            full((DH, DH)), full((1, DH)),
            full((DH, DOUT)), full((1, DOUT)),
            pl.BlockSpec((1, 1), lambda g: (0, 0),
                         memory_space=pltpu.MemorySpace.SMEM),
        ],
        out_specs=pl.BlockSpec((BBM, DOUT), lambda g: (g, 0)),
        out_shape=jax.ShapeDtypeStruct((B, DOUT), jnp.float32),
    )(xcn, xij,
      Wcn1, bcn1.reshape(1, DH), Wcn2, bcn2.reshape(1, DH),
      Wcn3, bcn3.reshape(1, DH),
      Wij1, bij1.reshape(1, DH), Wij2, bij2.reshape(1, DH),
      Wl1, bl1.reshape(1, DH), Wl2, bl2.reshape(1, DOUT),
      beta.reshape(1, 1))
    return out


# speedup vs baseline: 3.7258x; 3.7258x over previous
import functools

import jax
import jax.numpy as jnp
from jax import lax
from jax.experimental import pallas as pl
from jax.experimental.pallas import tpu as pltpu
from jax.experimental.pallas import tpu_sc as plsc

_NW = 32
_W = 96
_WC = _W // 16


def _sc_body(vs2_hbm, rp_hbm, tari_hbm, tarj_hbm, x2d_hbm, xflat_hbm,
             xcn_hbm, xij_hbm,
             tari_v, tarj_v, tmp_v, li_v, ri_v, lj_v, rj_v,
             wm, xcnbuf, xibuf, xjbuf, xrow,
             idxbuf, segjbuf, segibuf, widxbuf, wsegbuf,
             sem0, sem1, sem2,
             *, PW: int, DIN: int, NSENT: int):
    wid = lax.axis_index("s") * 2 + lax.axis_index("c")
    base = wid * PW
    ii = lax.iota(jnp.int32, 16)
    zeros16 = jnp.zeros((16,), jnp.int32)
    ones16 = jnp.ones((16,), jnp.int32)
    DC = DIN // 16

    pltpu.sync_copy(tari_hbm.at[pl.ds(base, PW)], tari_v)
    pltpu.sync_copy(tarj_hbm.at[pl.ds(base, PW)], tarj_v)
    pltpu.async_copy(rp_hbm.at[tari_v], li_v, sem0).wait()
    pltpu.async_copy(rp_hbm.at[tarj_v], lj_v, sem0).wait()

    def _plus1(c, buf_src, buf_dst):
        buf_dst[pl.ds(c * 16, 16)] = buf_src[pl.ds(c * 16, 16)] + 1

    lax.fori_loop(0, PW // 16, lambda c, _: (_plus1(c, tari_v, tmp_v), 0)[1], 0)
    pltpu.async_copy(rp_hbm.at[tmp_v], ri_v, sem0).wait()
    lax.fori_loop(0, PW // 16, lambda c, _: (_plus1(c, tarj_v, tmp_v), 0)[1], 0)
    pltpu.async_copy(rp_hbm.at[tmp_v], rj_v, sem0).wait()

    def _scal(ref, p):
        cb = (p // 16) * 16
        vec = ref[pl.ds(cb, 16)]
        return jnp.sum(jnp.where(ii == (p - cb), vec, 0))

    pltpu.async_copy(x2d_hbm.at[tari_v], xibuf, sem0)
    pltpu.async_copy(x2d_hbm.at[tarj_v], xjbuf, sem1)
    pltpu.make_async_copy(x2d_hbm.at[tari_v], xibuf, sem0).wait()
    pltpu.make_async_copy(x2d_hbm.at[tarj_v], xjbuf, sem1).wait()

    def _mul(t, _):
        p = t // DC
        c = t % DC
        xibuf[p, pl.ds(c * 16, 16)] = (xibuf[p, pl.ds(c * 16, 16)]
                                       * xjbuf[p, pl.ds(c * 16, 16)])
        return 0

    lax.fori_loop(0, PW * DC, _mul, 0)
    pltpu.sync_copy(xibuf, xij_hbm.at[pl.ds(base, PW)])

    def _zix(k, _):
        idxbuf[pl.ds(k * 16, 16)] = zeros16
        return 0

    lax.fori_loop(0, (4 * _W) // 16, _zix, 0)

    NWM = wm.shape[0]

    def _zwm(k, _):
        wm[pl.ds(k * 16, 16)] = zeros16
        return 0

    lax.fori_loop(0, NWM // 16, _zwm, 0)

    def _zxc(t, _):
        p = t // DC
        c = t % DC
        xcnbuf[p, pl.ds(c * 16, 16)] = jnp.zeros((16,), jnp.float32)
        return 0

    lax.fori_loop(0, PW * DC, _zxc, 0)

    def _fire(p):
        sl = (p % 2) * _W
        l_j = _scal(lj_v, p)
        r_j = _scal(rj_v, p)
        l_i = _scal(li_v, p)
        r_i = _scal(ri_v, p)
        def _bld(c, _):
            idxj = jnp.maximum(0, jnp.minimum(l_j + c * 16 + ii, r_j - 1))
            idxbuf[pl.ds(sl + c * 16, 16)] = idxj
            idxi = jnp.maximum(0, jnp.minimum(l_i + c * 16 + ii, r_i - 1))
            idxbuf[pl.ds(2 * _W + sl + c * 16, 16)] = idxi
            return 0

        lax.fori_loop(0, _WC, _bld, 0)
        pltpu.async_copy(vs2_hbm.at[idxbuf.at[pl.ds(sl, _W)]],
                         segjbuf.at[pl.ds(sl, _W)], sem2)
        pltpu.async_copy(vs2_hbm.at[idxbuf.at[pl.ds(2 * _W + sl, _W)]],
                         segibuf.at[pl.ds(sl, _W)], sem2)

    def _drain(p):
        sl = (p % 2) * _W
        pltpu.make_async_copy(vs2_hbm.at[idxbuf.at[pl.ds(sl, _W)]],
                              segjbuf.at[pl.ds(sl, _W)], sem2).wait()
        pltpu.make_async_copy(vs2_hbm.at[idxbuf.at[pl.ds(2 * _W + sl, _W)]],
                              segibuf.at[pl.ds(sl, _W)], sem2).wait()

    def _set_chunk(vj, m):
        plsc.store_scatter(wm, [vj], ones16, mask=m)

    def _clear_chunk(vj, m):
        plsc.store_scatter(wm, [vj], zeros16, mask=m)

    def _probe_chunk(p, vi, m):
        hit = plsc.load_gather(wm, [vi])
        hv = jnp.where(m, hit, 0)
        anyhit = jnp.max(hv)

        @pl.when(anyhit > 0)
        def _():
            def _lane(lane, _):
                w_l = jnp.sum(jnp.where(ii == lane, hv, 0))

                @pl.when(w_l > 0)
                def _():
                    v_l = jnp.sum(jnp.where(ii == lane, vi, 0))
                    off = pl.multiple_of(v_l * DIN, DIN)
                    pltpu.sync_copy(xflat_hbm.at[pl.ds(off, DIN)], xrow)

                    def _acc(c2, _):
                        xcnbuf[p, pl.ds(c2 * 16, 16)] = (
                            xcnbuf[p, pl.ds(c2 * 16, 16)]
                            + xrow[pl.ds(c2 * 16, 16)])
                        return 0

                    lax.fori_loop(0, DC, _acc, 0)
                return 0

            lax.fori_loop(0, 16, _lane, 0)

    def _extra_windows(l0, r, chunk_fn):
        def cond(off):
            return off < r

        def step(off):
            def _bw(c, _):
                idx = jnp.maximum(0, jnp.minimum(off + c * 16 + ii, r - 1))
                widxbuf[pl.ds(c * 16, 16)] = idx
                return 0

            lax.fori_loop(0, _WC, _bw, 0)
            pltpu.async_copy(vs2_hbm.at[widxbuf], wsegbuf, sem0).wait()

            def _pw(c, _):
                vec = wsegbuf[pl.ds(c * 16, 16)]
                m = (off + c * 16 + ii) < r
                chunk_fn(vec, m)
                return 0

            lax.fori_loop(0, _WC, _pw, 0)
            return off + _W

        lax.while_loop(cond, step, l0 + _W)

    _fire(0)

    def _pair(p, _):
        _drain(p)

        @pl.when(p + 1 < PW)
        def _():
            _fire(p + 1)

        sl = (p % 2) * _W
        l_j = _scal(lj_v, p)
        r_j = _scal(rj_v, p)
        l_i = _scal(li_v, p)
        r_i = _scal(ri_v, p)

        def _sc(c, _):
            vj = segjbuf[pl.ds(sl + c * 16, 16)]
            m = (l_j + c * 16 + ii) < r_j
            _set_chunk(vj, m)
            return 0

        lax.fori_loop(0, _WC, _sc, 0)

        @pl.when(l_j + _W < r_j)
        def _():
            _extra_windows(l_j, r_j, _set_chunk)

        def _pc(c, _):
            vi = segibuf[pl.ds(sl + c * 16, 16)]
            m = (l_i + c * 16 + ii) < r_i
            _probe_chunk(p, vi, m)
            return 0

        lax.fori_loop(0, _WC, _pc, 0)

        @pl.when(l_i + _W < r_i)
        def _():
            _extra_windows(l_i, r_i, lambda vec, m: _probe_chunk(p, vec, m))

        def _cc(c, _):
            vj = segjbuf[pl.ds(sl + c * 16, 16)]
            m = (l_j + c * 16 + ii) < r_j
            _clear_chunk(vj, m)
            return 0

        lax.fori_loop(0, _WC, _cc, 0)

        @pl.when(l_j + _W < r_j)
        def _():
            _extra_windows(l_j, r_j, _clear_chunk)

        return 0

    lax.fori_loop(0, PW, _pair, 0)

    pltpu.sync_copy(xcnbuf, xcn_hbm.at[pl.ds(base, PW)])


def _mlp_body(xcn_ref, xij_ref,
              Wcn1_r, bcn1_r, Wcn2_r, bcn2_r, Wcn3_r, bcn3_r,
              Wij1_r, bij1_r, Wij2_r, bij2_r,
              Wl1_r, bl1_r, Wl2_r, bl2_r, beta_r,
              out_ref):
    f32 = jnp.float32
    h = jnp.maximum(jnp.dot(xcn_ref[...], Wcn1_r[...],
                            preferred_element_type=f32) + bcn1_r[...], 0.0)
    h = jnp.maximum(jnp.dot(h, Wcn2_r[...],
                            preferred_element_type=f32) + bcn2_r[...], 0.0)
    hcn = jnp.dot(h, Wcn3_r[...], preferred_element_type=f32) + bcn3_r[...]

    hij = jnp.maximum(jnp.dot(xij_ref[...], Wij1_r[...],
                              preferred_element_type=f32) + bij1_r[...], 0.0)
    hij = jnp.dot(hij, Wij2_r[...], preferred_element_type=f32) + bij2_r[...]

    z = hcn * beta_r[0, 0] + hij
    o = jnp.maximum(jnp.dot(z, Wl1_r[...],
                            preferred_element_type=f32) + bl1_r[...], 0.0)
    out_ref[...] = (jnp.dot(o, Wl2_r[...], preferred_element_type=f32)
                    + bl2_r[0, 0])


def kernel(x, edge_index, tar_ei, beta, Wcn1, bcn1, Wcn2, bcn2, Wcn3, bcn3,
           Wij1, bij1, Wij2, bij2, Wl1, bl1, Wl2, bl2):
    N, DIN = x.shape
    E = edge_index.shape[1]
    B = tar_ei.shape[1]
    DH = Wcn1.shape[1]
    DOUT = Wl2.shape[1]
    PW = B // _NW

    M = 1
    while M < N:
        M *= 2
    e0 = edge_index[0].astype(jnp.int32)
    e1 = edge_index[1].astype(jnp.int32)
    keys = jnp.sort(e0 * M + e1)
    v = keys & (M - 1)
    uq = jnp.concatenate([jnp.ones((1,), bool), keys[1:] != keys[:-1]])
    vs2 = jnp.where(uq, v, N).astype(jnp.int32)
    deg = jnp.zeros((N,), jnp.int32).at[e0].add(1)
    rp = jnp.concatenate([jnp.zeros((1,), jnp.int32),
                          jnp.cumsum(deg).astype(jnp.int32)])
    tari = tar_ei[0].astype(jnp.int32)
    tarj = tar_ei[1].astype(jnp.int32)
    xflat = jnp.zeros(((N + 8) * DIN,), jnp.float32).at[:N * DIN].set(
        x.reshape(-1))
    NWM = ((N + 1 + 15) // 16) * 16

    sc = functools.partial(
        pl.kernel,
        mesh=plsc.VectorSubcoreMesh(core_axis_name="c", subcore_axis_name="s"),
        compiler_params=pltpu.CompilerParams(needs_layout_passes=False),
        out_type=[jax.ShapeDtypeStruct((B, DIN), jnp.float32),
                  jax.ShapeDtypeStruct((B, DIN), jnp.float32)],
        scratch_types=[
            pltpu.VMEM((PW,), jnp.int32),
            pltpu.VMEM((PW,), jnp.int32),
            pltpu.VMEM((PW,), jnp.int32),
            pltpu.VMEM((PW,), jnp.int32),
            pltpu.VMEM((PW,), jnp.int32),
            pltpu.VMEM((PW,), jnp.int32),
            pltpu.VMEM((PW,), jnp.int32),
            pltpu.VMEM((NWM,), jnp.int32),
            pltpu.VMEM((PW, DIN), jnp.float32),
            pltpu.VMEM((PW, DIN), jnp.float32),
            pltpu.VMEM((PW, DIN), jnp.float32),
            pltpu.VMEM((DIN,), jnp.float32),
            pltpu.VMEM((4 * _W,), jnp.int32),
            pltpu.VMEM((2 * _W,), jnp.int32),
            pltpu.VMEM((2 * _W,), jnp.int32),
            pltpu.VMEM((_W,), jnp.int32),
            pltpu.VMEM((_W,), jnp.int32),
            pltpu.SemaphoreType.DMA,
            pltpu.SemaphoreType.DMA,
            pltpu.SemaphoreType.DMA,
        ],
    )
    xcn, xij = sc(functools.partial(_sc_body, PW=PW, DIN=DIN, NSENT=N))(
        vs2, rp, tari, tarj, x, xflat)

    BBM = 512 if B % 512 == 0 else B
    full = lambda shape: pl.BlockSpec(shape, lambda g: (0, 0))
    out = pl.pallas_call(
        _mlp_body,
        grid=(B // BBM,),
        in_specs=[
            pl.BlockSpec((BBM, DIN), lambda g: (g, 0)),
            pl.BlockSpec((BBM, DIN), lambda g: (g, 0)),
            full((DIN, DH)), full((1, DH)),
            full((DH, DH)), full((1, DH)),
            full((DH, DH)), full((1, DH)),
            full((DIN, DH)), full((1, DH)),
            full((DH, DH)), full((1, DH)),
            full((DH, DH)), full((1, DH)),
            full((DH, DOUT)), full((1, DOUT)),
            pl.BlockSpec((1, 1), lambda g: (0, 0),
                         memory_space=pltpu.MemorySpace.SMEM),
        ],
        out_specs=pl.BlockSpec((BBM, DOUT), lambda g: (g, 0)),
        out_shape=jax.ShapeDtypeStruct((B, DOUT), jnp.float32),
    )(xcn, xij,
      Wcn1, bcn1.reshape(1, DH), Wcn2, bcn2.reshape(1, DH),
      Wcn3, bcn3.reshape(1, DH),
      Wij1, bij1.reshape(1, DH), Wij2, bij2.reshape(1, DH),
      Wl1, bl1.reshape(1, DH), Wl2, bl2.reshape(1, DOUT),
      beta.reshape(1, 1))
    return out
